# Initial kernel scaffold; baseline (speedup 1.0000x reference)
import jax
import jax.numpy as jnp
from jax.experimental import pallas as pl


def kernel(target, pred, latent, R_xyz):
    raise NotImplementedError("write your pallas kernel here")


# speedup vs baseline: 10.4228x; 10.4228x over previous
import functools

import jax
import jax.numpy as jnp
from jax.experimental import pallas as pl
from jax.experimental.pallas import tpu as pltpu

N_MAX_K = 12
ROWS, COLS = 4096, 8192
BLK_ROWS = 256
LAT_N = 1000000
LAT_R, LAT_C = 8192, 128
NEG_INF = float("-inf")


def _l1_body(t_ref, p_ref, o_ref):
    i = pl.program_id(0)

    @pl.when(i == 0)
    def _():
        o_ref[...] = jnp.zeros((1, 1), jnp.float32)

    o_ref[...] += jnp.sum(jnp.abs(t_ref[...] - p_ref[...])).reshape(1, 1)


def _topk_body(lat_ref, r_ref, l1_ref, o_ref):
    x = lat_ref[...]
    row_iota = jax.lax.broadcasted_iota(jnp.int32, (LAT_R, LAT_C), 0)
    col_iota = jax.lax.broadcasted_iota(jnp.int32, (LAT_R, LAT_C), 1)
    iota = row_iota * LAT_C + col_iota

    idxs = []
    for _ in range(N_MAX_K):
        m = jnp.max(x)
        idx = jnp.min(jnp.where(x == m, iota, jnp.int32(2147483647)))
        idxs.append(idx)
        x = jnp.where(iota == idx, NEG_INF, x)

    lane = jax.lax.broadcasted_iota(jnp.int32, (1, LAT_C), 1)
    coords = []
    for idx in idxs:
        row = idx // LAT_C
        col = idx - row * LAT_C
        slab = r_ref[:, pl.ds(row, 1), :]
        c = jnp.sum(jnp.where(lane == col, slab[:, 0, :], 0.0), axis=1)
        coords.append(c)
    cmat = jnp.stack(coords)

    diff = cmat[:, None, :] - cmat[None, :, :]
    d2 = jnp.sum(diff * diff, axis=-1)
    pd_sum = 0.5 * jnp.sum(jnp.sqrt(d2))

    total = l1_ref[0, 0] / (ROWS * COLS) + pd_sum / N_MAX_K
    o_ref[...] = total.reshape(1, 1)


@jax.jit
def kernel(target, pred, latent, R_xyz):
    l1_sum = pl.pallas_call(
        _l1_body,
        grid=(ROWS // BLK_ROWS,),
        in_specs=[
            pl.BlockSpec((BLK_ROWS, COLS), lambda i: (i, 0)),
            pl.BlockSpec((BLK_ROWS, COLS), lambda i: (i, 0)),
        ],
        out_specs=pl.BlockSpec((1, 1), lambda i: (0, 0)),
        out_shape=jax.ShapeDtypeStruct((1, 1), jnp.float32),
        compiler_params=pltpu.CompilerParams(
            dimension_semantics=("arbitrary",),
        ),
    )(target, pred)

    pad = LAT_R * LAT_C - LAT_N
    lat_p = jnp.pad(latent, (0, pad), constant_values=NEG_INF).reshape(LAT_R, LAT_C)
    r_p = jnp.pad(R_xyz, ((0, 0), (0, pad))).reshape(3, LAT_R, LAT_C)

    total = pl.pallas_call(
        _topk_body,
        in_specs=[
            pl.BlockSpec((LAT_R, LAT_C), lambda: (0, 0)),
            pl.BlockSpec((3, LAT_R, LAT_C), lambda: (0, 0, 0)),
            pl.BlockSpec((1, 1), lambda: (0, 0)),
        ],
        out_specs=pl.BlockSpec((1, 1), lambda: (0, 0)),
        out_shape=jax.ShapeDtypeStruct((1, 1), jnp.float32),
    )(lat_p, r_p, l1_sum)

    return total.reshape(())


# speedup vs baseline: 17.1056x; 1.6412x over previous
import jax
import jax.numpy as jnp
from jax.experimental import pallas as pl
from jax.experimental.pallas import tpu as pltpu

N_MAX_K = 12
ROWS, COLS = 4096, 8192
BLK_ROWS = 256
N_BLKS = ROWS // BLK_ROWS
LAT_N = 1000000
LAT_R, LAT_C = 8192, 128
GROUPS = 64
GROUP_ROWS = LAT_R // GROUPS
NEG_INF = float("-inf")
I32_MAX = 2147483647


def _l1_body(t_ref, p_ref, o_ref):
    s = jnp.sum(jnp.abs(t_ref[...] - p_ref[...]))
    lane = jax.lax.broadcasted_iota(jnp.int32, (1, 1, 128), 2)
    o_ref[...] = jnp.where(lane == 0, s, 0.0)


def _topk_body(lat_ref, r_ref, l1_ref, o_ref, csem, c_smem):
    x3 = lat_ref[...].reshape(GROUPS, GROUP_ROWS, LAT_C)
    gm = jnp.max(x3, axis=1)

    g_iota = jax.lax.broadcasted_iota(jnp.int32, (GROUPS, LAT_C), 0)
    srow = jax.lax.broadcasted_iota(jnp.int32, (GROUP_ROWS, LAT_C), 0)
    slane = jax.lax.broadcasted_iota(jnp.int32, (GROUP_ROWS, LAT_C), 1)

    removed = []
    for _ in range(N_MAX_K):
        m = jnp.max(gm)
        g_star = jnp.min(jnp.where(gm == m, g_iota, I32_MAX))
        base_row = g_star * GROUP_ROWS
        slab = lat_ref[pl.ds(base_row, GROUP_ROWS), :]
        sl_idx = (base_row + srow) * LAT_C + slane
        avail = slab == m
        for ridx in removed:
            avail &= sl_idx != ridx
        idx_k = jnp.min(jnp.where(avail, sl_idx, I32_MAX))
        removed.append(idx_k)
        rm = sl_idx == idx_k
        for ridx in removed[:-1]:
            rm |= sl_idx == ridx
        newcol = jnp.max(jnp.where(rm, NEG_INF, slab), axis=0)
        gm = jnp.where(g_iota == g_star,
                       jnp.broadcast_to(newcol[None, :], (GROUPS, LAT_C)), gm)

    copies = []
    subs = []
    for k, idx in enumerate(removed):
        base = (idx // 128) * 128
        subs.append(idx - base)
        cp = pltpu.make_async_copy(
            r_ref.at[:, pl.ds(base, 128)], c_smem.at[k], csem)
        cp.start()
        copies.append(cp)
    for cp in copies:
        cp.wait()

    r16 = jax.lax.broadcasted_iota(jnp.int32, (16, 128), 0)
    c16 = jax.lax.broadcasted_iota(jnp.int32, (16, 128), 1)
    zero = jnp.zeros((16, 128), jnp.float32)
    a = [zero, zero, zero]
    b = [zero, zero, zero]
    for k in range(N_MAX_K):
        for d in range(3):
            v = c_smem[k, d, subs[k]]
            a[d] = jnp.where(r16 == k, v, a[d])
            b[d] = jnp.where(c16 == k, v, b[d])
    d2 = ((a[0] - b[0]) ** 2 + (a[1] - b[1]) ** 2 + (a[2] - b[2]) ** 2)
    valid = (r16 < N_MAX_K) & (c16 < N_MAX_K)
    pd_sum = 0.5 * jnp.sum(jnp.where(valid, jnp.sqrt(d2), 0.0))

    l1_total = l1_ref[0, 0, 0]
    for i in range(1, N_BLKS):
        l1_total += l1_ref[i, 0, 0]

    o_ref[0, 0] = l1_total / (ROWS * COLS) + pd_sum / N_MAX_K


@jax.jit
def kernel(target, pred, latent, R_xyz):
    l1_parts = pl.pallas_call(
        _l1_body,
        grid=(N_BLKS,),
        in_specs=[
            pl.BlockSpec((BLK_ROWS, COLS), lambda i: (i, 0)),
            pl.BlockSpec((BLK_ROWS, COLS), lambda i: (i, 0)),
        ],
        out_specs=pl.BlockSpec((1, 1, 128), lambda i: (i, 0, 0)),
        out_shape=jax.ShapeDtypeStruct((N_BLKS, 1, 128), jnp.float32),
        compiler_params=pltpu.CompilerParams(
            dimension_semantics=("parallel",),
        ),
    )(target, pred)

    pad = LAT_R * LAT_C - LAT_N
    lat_p = jnp.pad(latent, (0, pad), constant_values=NEG_INF).reshape(LAT_R, LAT_C)

    total = pl.pallas_call(
        _topk_body,
        in_specs=[
            pl.BlockSpec((LAT_R, LAT_C), lambda: (0, 0)),
            pl.BlockSpec(memory_space=pl.ANY),
            pl.BlockSpec(memory_space=pltpu.SMEM),
        ],
        out_specs=pl.BlockSpec(memory_space=pltpu.SMEM),
        out_shape=jax.ShapeDtypeStruct((1, 1), jnp.float32),
        scratch_shapes=[
            pltpu.SemaphoreType.DMA,
            pltpu.SMEM((N_MAX_K, 3, 128), jnp.float32),
        ],
    )(lat_p, R_xyz, l1_parts)

    return total.reshape(())
